# baseline (speedup 1.0000x reference)
import functools

import jax
import jax.numpy as jnp
from jax import lax
from jax.experimental import pallas as pl
from jax.experimental.pallas import tpu as pltpu
from jax.experimental.pallas import tpu_sc as plsc

B = 4
S = 2048
D = 2048
E = 8
K = 8
L = 1000

NC = 2
NS = 16
LANES = 16
NW = NC * NS

SBLK = 1024
DB = 1024
NEG = -3.0e38


def _gate_topk_body(w_ref, xa_ref, xb_ref, idx_ref, wout_ref, sc_ref):
    b = pl.program_id(0)
    nt = (((1,), (1,)), ((), ()))
    sc_ref[b, :, :SBLK] = lax.dot_general(
        w_ref[...], xa_ref[0], dimension_numbers=nt,
        preferred_element_type=jnp.float32)
    sc_ref[b, :, SBLK:] = lax.dot_general(
        w_ref[...], xb_ref[0], dimension_numbers=nt,
        preferred_element_type=jnp.float32)

    @pl.when(b == B - 1)
    def _topk():
        sc = jnp.reshape(sc_ref[...], (NW, S))
        m = jnp.max(sc, axis=1, keepdims=True)
        z = jnp.sum(jnp.exp(sc - m), axis=1, keepdims=True)
        iota = lax.broadcasted_iota(jnp.int32, (NW, S), 1)
        work = sc
        vals = []
        idxs = []
        for _ in range(K):
            mk = jnp.max(work, axis=1, keepdims=True)
            ik = jnp.min(jnp.where(work == mk, iota, S), axis=1, keepdims=True)
            vals.append(mk)
            idxs.append(ik)
            work = jnp.where(iota == ik, NEG, work)
        p = jnp.concatenate([jnp.exp(v - m) / z for v in vals], axis=1)
        pm = jnp.max(p, axis=1, keepdims=True)
        ev = jnp.exp(p - pm)
        es = jnp.sum(ev, axis=1, keepdims=True)
        w = ev / (E * es)
        gidx = jnp.concatenate(idxs, axis=1)
        boff = (lax.broadcasted_iota(jnp.int32, (NW, K), 0) // E) * S
        idx_ref[...] = gidx + boff
        wout_ref[...] = jnp.broadcast_to(w[:, :, None], (NW, K, LANES))


def _gate_topk(features, gate_W):
    return pl.pallas_call(
        _gate_topk_body,
        grid=(B,),
        in_specs=[
            pl.BlockSpec((E, D), lambda b: (0, 0)),
            pl.BlockSpec((1, SBLK, D), lambda b: (b, 0, 0)),
            pl.BlockSpec((1, SBLK, D), lambda b: (b, 1, 0)),
        ],
        out_specs=[
            pl.BlockSpec((NW, K), lambda b: (0, 0)),
            pl.BlockSpec((NW, K, LANES), lambda b: (0, 0, 0)),
        ],
        out_shape=[
            jax.ShapeDtypeStruct((NW, K), jnp.int32),
            jax.ShapeDtypeStruct((NW, K, LANES), jnp.float32),
        ],
        scratch_shapes=[pltpu.VMEM((B, E, S), jnp.float32)],
    )(gate_W, features, features)


_sc_mesh = plsc.VectorSubcoreMesh(core_axis_name="c", subcore_axis_name="s")


@functools.partial(
    pl.kernel,
    mesh=_sc_mesh,
    out_type=jax.ShapeDtypeStruct((NW, D), jnp.float32),
    scratch_types=[
        pltpu.VMEM((K,), jnp.int32),
        pltpu.VMEM((K, LANES), jnp.float32),
        pltpu.VMEM((K, D), jnp.float32),
        pltpu.VMEM((D,), jnp.float32),
        pltpu.SemaphoreType.DMA,
    ],
)
def _gather_combine(idx_hbm, w_hbm, feat_hbm, v_hbm,
                    idx_v, w_v, rows_v, out_v, sem):
    wid = lax.axis_index("s") * NC + lax.axis_index("c")
    e = wid // B
    b = wid - e * B
    j = b * E + e
    ca = pltpu.async_copy(idx_hbm.at[j], idx_v, sem)
    cb = pltpu.async_copy(w_hbm.at[j], w_v, sem)
    ca.wait()
    cb.wait()
    pltpu.async_copy(feat_hbm.at[idx_v], rows_v, sem).wait()
    ws = [w_v[k, :] for k in range(K)]

    def chunk(c, carry):
        base = c * LANES
        acc = rows_v[0, pl.ds(base, LANES)] * ws[0]
        for k in range(1, K):
            acc = acc + rows_v[k, pl.ds(base, LANES)] * ws[k]
        out_v[pl.ds(base, LANES)] = acc
        return carry

    lax.fori_loop(0, D // LANES, chunk, 0)
    pltpu.sync_copy(out_v, v_hbm.at[wid])


def _cls_body(v_ref, wa_ref, wb_ref, b_ref, out_ref):
    e = pl.program_id(0)

    @pl.when(e == 0)
    def _init():
        bias = (jnp.sum(b_ref[...], axis=0) * (1.0 / E))[None, :]
        out_ref[...] = jnp.broadcast_to(bias, (B, L))

    acc = lax.dot_general(
        v_ref[0, :, :DB], wa_ref[0],
        dimension_numbers=(((1,), (1,)), ((), ())),
        preferred_element_type=jnp.float32)
    acc += lax.dot_general(
        v_ref[0, :, DB:], wb_ref[0],
        dimension_numbers=(((1,), (1,)), ((), ())),
        preferred_element_type=jnp.float32)
    out_ref[...] += acc


def _classifier(v, cls_W, cls_b):
    return pl.pallas_call(
        _cls_body,
        grid=(E,),
        in_specs=[
            pl.BlockSpec((1, B, D), lambda e: (e, 0, 0)),
            pl.BlockSpec((1, L, DB), lambda e: (e, 0, 0)),
            pl.BlockSpec((1, L, DB), lambda e: (e, 0, 1)),
            pl.BlockSpec((E, L), lambda e: (0, 0)),
        ],
        out_specs=pl.BlockSpec((B, L), lambda e: (0, 0)),
        out_shape=jax.ShapeDtypeStruct((B, L), jnp.float32),
    )(v, cls_W, cls_W, cls_b)


def kernel(features, gate_W, gate_b, cls_W, cls_b):
    del gate_b
    idx_sc, w_sc = _gate_topk(features, gate_W)
    v = _gather_combine(idx_sc, w_sc, features.reshape(B * S, D))
    return _classifier(v.reshape(E, B, D), cls_W, cls_b)


# speedup vs baseline: 1.0321x; 1.0321x over previous
import functools

import jax
import jax.numpy as jnp
from jax import lax
from jax.experimental import pallas as pl
from jax.experimental.pallas import tpu as pltpu
from jax.experimental.pallas import tpu_sc as plsc

B = 4
S = 2048
D = 2048
E = 8
K = 8
L = 1000

NC = 2
NS = 16
LANES = 16
NW = NC * NS

SBLK = 512
DB = 512
NEG = -3.0e38


def _gate_topk_body(w_ref, xa_ref, xb_ref, xc_ref, xd_ref,
                    idx_ref, wout_ref, sc_ref):
    b = pl.program_id(0)
    nt = (((1,), (1,)), ((), ()))
    for i, x_ref in enumerate((xa_ref, xb_ref, xc_ref, xd_ref)):
        sc_ref[b, :, pl.ds(i * SBLK, SBLK)] = lax.dot_general(
            w_ref[...], x_ref[0], dimension_numbers=nt,
            preferred_element_type=jnp.float32)

    @pl.when(b == B - 1)
    def _topk():
        sc = jnp.reshape(sc_ref[...], (NW, S))
        m = jnp.max(sc, axis=1, keepdims=True)
        z = jnp.sum(jnp.exp(sc - m), axis=1, keepdims=True)
        iota = lax.broadcasted_iota(jnp.int32, (NW, S), 1)
        work = sc
        vals = []
        idxs = []
        for _ in range(K):
            mk = jnp.max(work, axis=1, keepdims=True)
            ik = jnp.min(jnp.where(work == mk, iota, S), axis=1, keepdims=True)
            vals.append(mk)
            idxs.append(ik)
            work = jnp.where(iota == ik, NEG, work)
        p = jnp.concatenate([jnp.exp(v - m) / z for v in vals], axis=1)
        pm = jnp.max(p, axis=1, keepdims=True)
        ev = jnp.exp(p - pm)
        es = jnp.sum(ev, axis=1, keepdims=True)
        w = ev / (E * es)
        gidx = jnp.concatenate(idxs, axis=1)
        boff = (lax.broadcasted_iota(jnp.int32, (NW, K), 0) // E) * S
        idx_ref[...] = gidx + boff
        wout_ref[...] = jnp.broadcast_to(w[:, :, None], (NW, K, LANES))


def _gate_topk(features, gate_W):
    return pl.pallas_call(
        _gate_topk_body,
        grid=(B,),
        in_specs=[
            pl.BlockSpec((E, D), lambda b: (0, 0)),
            pl.BlockSpec((1, SBLK, D), lambda b: (b, 0, 0)),
            pl.BlockSpec((1, SBLK, D), lambda b: (b, 1, 0)),
            pl.BlockSpec((1, SBLK, D), lambda b: (b, 2, 0)),
            pl.BlockSpec((1, SBLK, D), lambda b: (b, 3, 0)),
        ],
        out_specs=[
            pl.BlockSpec((NW, K), lambda b: (0, 0)),
            pl.BlockSpec((NW, K, LANES), lambda b: (0, 0, 0)),
        ],
        out_shape=[
            jax.ShapeDtypeStruct((NW, K), jnp.int32),
            jax.ShapeDtypeStruct((NW, K, LANES), jnp.float32),
        ],
        scratch_shapes=[pltpu.VMEM((B, E, S), jnp.float32)],
    )(gate_W, features, features, features, features)


_sc_mesh = plsc.VectorSubcoreMesh(core_axis_name="c", subcore_axis_name="s")


@functools.partial(
    pl.kernel,
    mesh=_sc_mesh,
    out_type=jax.ShapeDtypeStruct((NW, D), jnp.float32),
    scratch_types=[
        pltpu.VMEM((K,), jnp.int32),
        pltpu.VMEM((K, LANES), jnp.float32),
        pltpu.VMEM((K, D), jnp.float32),
        pltpu.VMEM((D,), jnp.float32),
        pltpu.SemaphoreType.DMA,
    ],
)
def _gather_combine(idx_hbm, w_hbm, feat_hbm, v_hbm,
                    idx_v, w_v, rows_v, out_v, sem):
    wid = lax.axis_index("s") * NC + lax.axis_index("c")
    e = wid // B
    b = wid - e * B
    j = b * E + e
    ca = pltpu.async_copy(idx_hbm.at[j], idx_v, sem)
    cb = pltpu.async_copy(w_hbm.at[j], w_v, sem)
    ca.wait()
    cb.wait()
    pltpu.async_copy(feat_hbm.at[idx_v], rows_v, sem).wait()
    ws = [w_v[k, :] for k in range(K)]

    def chunk(c, carry):
        base = c * LANES
        acc = rows_v[0, pl.ds(base, LANES)] * ws[0]
        for k in range(1, K):
            acc = acc + rows_v[k, pl.ds(base, LANES)] * ws[k]
        out_v[pl.ds(base, LANES)] = acc
        return carry

    lax.fori_loop(0, D // LANES, chunk, 0)
    pltpu.sync_copy(out_v, v_hbm.at[wid])


def _cls_body(v_ref, wa_ref, wb_ref, wc_ref, wd_ref, b_ref, out_ref):
    e = pl.program_id(0)

    @pl.when(e == 0)
    def _init():
        bias = (jnp.sum(b_ref[...], axis=0) * (1.0 / E))[None, :]
        out_ref[...] = jnp.broadcast_to(bias, (B, L))

    nt = (((1,), (1,)), ((), ()))
    acc = lax.dot_general(v_ref[0, :, :DB], wa_ref[0], dimension_numbers=nt,
                          preferred_element_type=jnp.float32)
    acc += lax.dot_general(v_ref[0, :, DB:2 * DB], wb_ref[0],
                           dimension_numbers=nt,
                           preferred_element_type=jnp.float32)
    acc += lax.dot_general(v_ref[0, :, 2 * DB:3 * DB], wc_ref[0],
                           dimension_numbers=nt,
                           preferred_element_type=jnp.float32)
    acc += lax.dot_general(v_ref[0, :, 3 * DB:], wd_ref[0],
                           dimension_numbers=nt,
                           preferred_element_type=jnp.float32)
    out_ref[...] += acc


def _classifier(v, cls_W, cls_b):
    return pl.pallas_call(
        _cls_body,
        grid=(E,),
        in_specs=[
            pl.BlockSpec((1, B, D), lambda e: (e, 0, 0)),
            pl.BlockSpec((1, L, DB), lambda e: (e, 0, 0)),
            pl.BlockSpec((1, L, DB), lambda e: (e, 0, 1)),
            pl.BlockSpec((1, L, DB), lambda e: (e, 0, 2)),
            pl.BlockSpec((1, L, DB), lambda e: (e, 0, 3)),
            pl.BlockSpec((E, L), lambda e: (0, 0)),
        ],
        out_specs=pl.BlockSpec((B, L), lambda e: (0, 0)),
        out_shape=jax.ShapeDtypeStruct((B, L), jnp.float32),
    )(v, cls_W, cls_W, cls_W, cls_W, cls_b)


def kernel(features, gate_W, gate_b, cls_W, cls_b):
    del gate_b
    idx_sc, w_sc = _gate_topk(features, gate_W)
    v = _gather_combine(idx_sc, w_sc, features.reshape(B * S, D))
    return _classifier(v.reshape(E, B, D), cls_W, cls_b)
